# Initial kernel scaffold; baseline (speedup 1.0000x reference)
import jax
import jax.numpy as jnp
from jax.experimental import pallas as pl


def kernel(pos_h, pos_t, pos_r, neg_h, neg_t, neg_r, ent_emb, rel_emb):
    raise NotImplementedError("write your pallas kernel here")


# speedup vs baseline: 1.7761x; 1.7761x over previous
import functools

import jax
import jax.numpy as jnp
from jax import lax
from jax.experimental import pallas as pl
from jax.experimental.pallas import tpu as pltpu
from jax.experimental.pallas import tpu_sc as plsc

D = 128
B = 16384
L = 16

_info = plsc.get_sparse_core_info()
_NC, _NS = _info.num_cores, _info.num_subcores
NW = _NC * _NS
ROWS_PER_W = B // NW
CHUNK = 128
NCHUNK = ROWS_PER_W // CHUNK
GROUPS = CHUNK // L


def _make_kernel():
    mesh = plsc.VectorSubcoreMesh(core_axis_name="c", subcore_axis_name="s")

    @functools.partial(
        pl.kernel,
        mesh=mesh,
        compiler_params=pltpu.CompilerParams(needs_layout_passes=False),
        out_type=(
            jax.ShapeDtypeStruct((B,), jnp.float32),
            jax.ShapeDtypeStruct((B,), jnp.float32),
        ),
        scratch_types=[
            pltpu.VMEM((ROWS_PER_W,), jnp.int32),
            pltpu.VMEM((ROWS_PER_W,), jnp.int32),
            pltpu.VMEM((ROWS_PER_W,), jnp.int32),
            pltpu.VMEM((CHUNK, D), jnp.float32),
            pltpu.VMEM((CHUNK, D), jnp.float32),
            pltpu.VMEM((CHUNK, D), jnp.float32),
            pltpu.VMEM((ROWS_PER_W,), jnp.float32),
            pltpu.SemaphoreType.DMA,
        ],
    )
    def trans_e(pos_h, pos_t, pos_r, neg_h, neg_t, neg_r, ent_emb, rel_emb,
                pos_out, neg_out,
                idxh_v, idxt_v, idxr_v, h_v, t_v, r_v, out_v, sem):
        wid = lax.axis_index("s") * _NC + lax.axis_index("c")
        base = wid * ROWS_PER_W
        lane_ids = lax.iota(jnp.int32, L)
        for ih, it, ir, out_hbm in (
            (pos_h, pos_t, pos_r, pos_out),
            (neg_h, neg_t, neg_r, neg_out),
        ):
            pltpu.sync_copy(ih.at[pl.ds(base, ROWS_PER_W)], idxh_v)
            pltpu.sync_copy(it.at[pl.ds(base, ROWS_PER_W)], idxt_v)
            pltpu.sync_copy(ir.at[pl.ds(base, ROWS_PER_W)], idxr_v)

            def chunk_body(c, _):
                off = c * CHUNK
                cp_h = pltpu.async_copy(
                    ent_emb.at[idxh_v.at[pl.ds(off, CHUNK)]], h_v, sem)
                cp_t = pltpu.async_copy(
                    ent_emb.at[idxt_v.at[pl.ds(off, CHUNK)]], t_v, sem)
                cp_r = pltpu.async_copy(
                    rel_emb.at[idxr_v.at[pl.ds(off, CHUNK)]], r_v, sem)
                cp_h.wait()
                cp_t.wait()
                cp_r.wait()

                def group_body(g, _):
                    rbase = g * L
                    total = jnp.zeros((L,), jnp.float32)
                    for rr in range(L):
                        row = rbase + rr
                        acc = jnp.zeros((L,), jnp.float32)
                        for j in range(D // L):
                            h = h_v[row, pl.ds(j * L, L)]
                            t = t_v[row, pl.ds(j * L, L)]
                            r = r_v[row, pl.ds(j * L, L)]
                            acc = acc + jnp.abs(h + r - t)
                        total = jnp.where(lane_ids == rr, jnp.sum(acc), total)
                    out_v[pl.ds(off + rbase, L)] = total
                    return 0

                lax.fori_loop(0, GROUPS, group_body, 0)
                return 0

            lax.fori_loop(0, NCHUNK, chunk_body, 0)
            pltpu.sync_copy(out_v, out_hbm.at[pl.ds(base, ROWS_PER_W)])

    return trans_e


_trans_e = _make_kernel()


@jax.jit
def kernel(pos_h, pos_t, pos_r, neg_h, neg_t, neg_r, ent_emb, rel_emb):
    return _trans_e(pos_h, pos_t, pos_r, neg_h, neg_t, neg_r, ent_emb, rel_emb)


# speedup vs baseline: 1.9592x; 1.1031x over previous
import functools

import jax
import jax.numpy as jnp
from jax import lax
from jax.experimental import pallas as pl
from jax.experimental.pallas import tpu as pltpu
from jax.experimental.pallas import tpu_sc as plsc

D = 128
B = 16384
L = 16

_info = plsc.get_sparse_core_info()
_NC, _NS = _info.num_cores, _info.num_subcores
NW = _NC * _NS
ROWS_PER_W = B // NW
TOT_ROWS = 2 * ROWS_PER_W
CHUNK = 128
NITER = TOT_ROWS // CHUNK
GROUPS = CHUNK // L
NBUF = 2


def _make_kernel():
    mesh = plsc.VectorSubcoreMesh(core_axis_name="c", subcore_axis_name="s")

    @functools.partial(
        pl.kernel,
        mesh=mesh,
        compiler_params=pltpu.CompilerParams(needs_layout_passes=False),
        out_type=(
            jax.ShapeDtypeStruct((B,), jnp.float32),
            jax.ShapeDtypeStruct((B,), jnp.float32),
        ),
        scratch_types=[
            pltpu.VMEM((TOT_ROWS,), jnp.int32),
            pltpu.VMEM((TOT_ROWS,), jnp.int32),
            pltpu.VMEM((TOT_ROWS,), jnp.int32),
            pltpu.VMEM((NBUF, CHUNK, D), jnp.float32),
            pltpu.VMEM((NBUF, CHUNK, D), jnp.float32),
            pltpu.VMEM((NBUF, CHUNK, D), jnp.float32),
            pltpu.VMEM((TOT_ROWS,), jnp.float32),
            [pltpu.SemaphoreType.DMA] * NBUF,
        ],
    )
    def trans_e(pos_h, pos_t, pos_r, neg_h, neg_t, neg_r, ent_emb, rel_emb,
                pos_out, neg_out,
                idxh_v, idxt_v, idxr_v, h_v, t_v, r_v, out_v, sems):
        wid = lax.axis_index("s") * _NC + lax.axis_index("c")
        base = wid * ROWS_PER_W
        lane_ids = lax.iota(jnp.int32, L)

        pltpu.sync_copy(pos_h.at[pl.ds(base, ROWS_PER_W)],
                        idxh_v.at[pl.ds(0, ROWS_PER_W)])
        pltpu.sync_copy(neg_h.at[pl.ds(base, ROWS_PER_W)],
                        idxh_v.at[pl.ds(ROWS_PER_W, ROWS_PER_W)])
        pltpu.sync_copy(pos_t.at[pl.ds(base, ROWS_PER_W)],
                        idxt_v.at[pl.ds(0, ROWS_PER_W)])
        pltpu.sync_copy(neg_t.at[pl.ds(base, ROWS_PER_W)],
                        idxt_v.at[pl.ds(ROWS_PER_W, ROWS_PER_W)])
        pltpu.sync_copy(pos_r.at[pl.ds(base, ROWS_PER_W)],
                        idxr_v.at[pl.ds(0, ROWS_PER_W)])
        pltpu.sync_copy(neg_r.at[pl.ds(base, ROWS_PER_W)],
                        idxr_v.at[pl.ds(ROWS_PER_W, ROWS_PER_W)])

        def fire(i, b):
            off = i * CHUNK
            pltpu.async_copy(
                ent_emb.at[idxh_v.at[pl.ds(off, CHUNK)]], h_v.at[b], sems[b])
            pltpu.async_copy(
                ent_emb.at[idxt_v.at[pl.ds(off, CHUNK)]], t_v.at[b], sems[b])
            pltpu.async_copy(
                rel_emb.at[idxr_v.at[pl.ds(off, CHUNK)]], r_v.at[b], sems[b])

        def drain(b):
            pltpu.make_async_copy(
                ent_emb.at[pl.ds(0, CHUNK)], h_v.at[b], sems[b]).wait()
            pltpu.make_async_copy(
                ent_emb.at[pl.ds(0, CHUNK)], t_v.at[b], sems[b]).wait()
            pltpu.make_async_copy(
                ent_emb.at[pl.ds(0, CHUNK)], r_v.at[b], sems[b]).wait()

        for b in range(NBUF):
            fire(b, b)

        def pair_body(p, _):
            for b in range(NBUF):
                i = p * NBUF + b
                drain(b)

                def group_body(g, _, b=b, i=i):
                    total = jnp.zeros((L,), jnp.float32)
                    for rr in range(L):
                        acc = jnp.zeros((L,), jnp.float32)
                        for j in range(D // L):
                            h = h_v[b, g * L + rr, pl.ds(j * L, L)]
                            t = t_v[b, g * L + rr, pl.ds(j * L, L)]
                            r = r_v[b, g * L + rr, pl.ds(j * L, L)]
                            acc = acc + jnp.abs(h + r - t)
                        total = jnp.where(lane_ids == rr, jnp.sum(acc), total)
                    out_v[pl.ds(i * CHUNK + g * L, L)] = total
                    return 0

                lax.fori_loop(0, GROUPS, group_body, 0)

                @pl.when(i + NBUF < NITER)
                def _fire_next(b=b, i=i):
                    fire(i + NBUF, b)
            return 0

        lax.fori_loop(0, NITER // NBUF, pair_body, 0)

        pltpu.sync_copy(out_v.at[pl.ds(0, ROWS_PER_W)],
                        pos_out.at[pl.ds(base, ROWS_PER_W)])
        pltpu.sync_copy(out_v.at[pl.ds(ROWS_PER_W, ROWS_PER_W)],
                        neg_out.at[pl.ds(base, ROWS_PER_W)])

    return trans_e


_trans_e = _make_kernel()


@jax.jit
def kernel(pos_h, pos_t, pos_r, neg_h, neg_t, neg_r, ent_emb, rel_emb):
    return _trans_e(pos_h, pos_t, pos_r, neg_h, neg_t, neg_r, ent_emb, rel_emb)


# speedup vs baseline: 3.4564x; 1.7641x over previous
import functools

import jax
import jax.numpy as jnp
from jax import lax
from jax.experimental import pallas as pl
from jax.experimental.pallas import tpu as pltpu
from jax.experimental.pallas import tpu_sc as plsc

D = 128
B = 16384
L = 16

_info = plsc.get_sparse_core_info()
_NC, _NS = _info.num_cores, _info.num_subcores
NW = _NC * _NS
ROWS_PER_W = B // NW
TOT_ROWS = 2 * ROWS_PER_W
CHUNK = 128
NITER = TOT_ROWS // CHUNK
GROUPS = CHUNK // L
NBUF = 2
RBLK = 4


def _make_kernel():
    mesh = plsc.VectorSubcoreMesh(core_axis_name="c", subcore_axis_name="s")

    @functools.partial(
        pl.kernel,
        mesh=mesh,
        compiler_params=pltpu.CompilerParams(needs_layout_passes=False),
        out_type=(
            jax.ShapeDtypeStruct((B,), jnp.float32),
            jax.ShapeDtypeStruct((B,), jnp.float32),
        ),
        scratch_types=[
            pltpu.VMEM((TOT_ROWS,), jnp.int32),
            pltpu.VMEM((TOT_ROWS,), jnp.int32),
            pltpu.VMEM((TOT_ROWS,), jnp.int32),
            pltpu.VMEM((NBUF, CHUNK, D), jnp.float32),
            pltpu.VMEM((NBUF, CHUNK, D), jnp.float32),
            pltpu.VMEM((TOT_ROWS,), jnp.float32),
            [pltpu.SemaphoreType.DMA] * NBUF,
            [pltpu.SemaphoreType.DMA] * NBUF,
            [pltpu.SemaphoreType.DMA] * NBUF,
        ],
    )
    def trans_e(pos_h, pos_t, pos_r, neg_h, neg_t, neg_r, ent_emb, rel_emb,
                pos_out, neg_out,
                idxh_v, idxt_v, idxr_v, h_v, t_v, out_v,
                sems_h, sems_t, sems_r):
        wid = lax.axis_index("s") * _NC + lax.axis_index("c")
        base = wid * ROWS_PER_W
        lane_ids = lax.iota(jnp.int32, L)

        pltpu.sync_copy(pos_h.at[pl.ds(base, ROWS_PER_W)],
                        idxh_v.at[pl.ds(0, ROWS_PER_W)])
        pltpu.sync_copy(neg_h.at[pl.ds(base, ROWS_PER_W)],
                        idxh_v.at[pl.ds(ROWS_PER_W, ROWS_PER_W)])
        pltpu.sync_copy(pos_t.at[pl.ds(base, ROWS_PER_W)],
                        idxt_v.at[pl.ds(0, ROWS_PER_W)])
        pltpu.sync_copy(neg_t.at[pl.ds(base, ROWS_PER_W)],
                        idxt_v.at[pl.ds(ROWS_PER_W, ROWS_PER_W)])
        pltpu.sync_copy(pos_r.at[pl.ds(base, ROWS_PER_W)],
                        idxr_v.at[pl.ds(0, ROWS_PER_W)])
        pltpu.sync_copy(neg_r.at[pl.ds(base, ROWS_PER_W)],
                        idxr_v.at[pl.ds(ROWS_PER_W, ROWS_PER_W)])

        def fire_ht(i, b):
            off = i * CHUNK
            pltpu.async_copy(
                ent_emb.at[idxh_v.at[pl.ds(off, CHUNK)]], h_v.at[b],
                sems_h[b])
            pltpu.async_copy(
                ent_emb.at[idxt_v.at[pl.ds(off, CHUNK)]], t_v.at[b],
                sems_t[b])

        def fire_radd(i, b):
            off = i * CHUNK
            pltpu.async_copy(
                rel_emb.at[idxr_v.at[pl.ds(off, CHUNK)]], h_v.at[b],
                sems_r[b], add=True)

        def wait_h(b):
            pltpu.make_async_copy(
                ent_emb.at[pl.ds(0, CHUNK)], h_v.at[b], sems_h[b]).wait()

        def wait_t(b):
            pltpu.make_async_copy(
                ent_emb.at[pl.ds(0, CHUNK)], t_v.at[b], sems_t[b]).wait()

        def wait_r(b):
            pltpu.make_async_copy(
                ent_emb.at[pl.ds(0, CHUNK)], h_v.at[b], sems_r[b]).wait()

        fire_ht(0, 0)
        wait_h(0)
        fire_radd(0, 0)
        fire_ht(1, 1)

        def pair_body(p, _):
            for b in range(NBUF):
                i = p * NBUF + b
                b1 = (b + 1) % NBUF

                @pl.when(i + 1 < NITER)
                def _radd_next(i=i, b1=b1):
                    wait_h(b1)
                    fire_radd(i + 1, b1)

                wait_t(b)
                wait_r(b)

                def group_body(g, total_blk, b=b, i=i):
                    del total_blk

                    def block_body(k, total, b=b, g=g):
                        for kk in range(RBLK):
                            rr = k * RBLK + kk
                            acc = jnp.zeros((L,), jnp.float32)
                            for j in range(D // L):
                                hr = h_v[b, g * L + rr, pl.ds(j * L, L)]
                                t = t_v[b, g * L + rr, pl.ds(j * L, L)]
                                acc = acc + jnp.abs(hr - t)
                            total = jnp.where(
                                lane_ids == rr, jnp.sum(acc), total)
                        return total

                    total = lax.fori_loop(
                        0, L // RBLK, block_body, jnp.zeros((L,), jnp.float32))
                    out_v[pl.ds(i * CHUNK + g * L, L)] = total
                    return 0

                lax.fori_loop(0, GROUPS, group_body, 0)

                @pl.when(i + NBUF < NITER)
                def _fire_next(b=b, i=i):
                    fire_ht(i + NBUF, b)
            return 0

        lax.fori_loop(0, NITER // NBUF, pair_body, 0)

        pltpu.sync_copy(out_v.at[pl.ds(0, ROWS_PER_W)],
                        pos_out.at[pl.ds(base, ROWS_PER_W)])
        pltpu.sync_copy(out_v.at[pl.ds(ROWS_PER_W, ROWS_PER_W)],
                        neg_out.at[pl.ds(base, ROWS_PER_W)])

    return trans_e


_trans_e = _make_kernel()


@jax.jit
def kernel(pos_h, pos_t, pos_r, neg_h, neg_t, neg_r, ent_emb, rel_emb):
    return _trans_e(pos_h, pos_t, pos_r, neg_h, neg_t, neg_r, ent_emb, rel_emb)
